# Initial kernel scaffold; baseline (speedup 1.0000x reference)
import jax
import jax.numpy as jnp
from jax.experimental import pallas as pl


def kernel(x, W_E):
    raise NotImplementedError("write your pallas kernel here")


# speedup vs baseline: 1.3068x; 1.3068x over previous
import functools

import jax
import jax.numpy as jnp
from jax import lax
from jax.experimental import pallas as pl
from jax.experimental.pallas import tpu as pltpu
from jax.experimental.pallas import tpu_sc as plsc


_info = plsc.get_sparse_core_info()
_NC, _NS = _info.num_cores, _info.num_subcores
_NW = _NC * _NS

_CH = 128


def _embed_gather(table, idx3, B, D, n_ch):
    mesh = plsc.VectorSubcoreMesh(core_axis_name="c", subcore_axis_name="s")
    b_per_w = n_ch * _CH

    @functools.partial(
        pl.kernel,
        mesh=mesh,
        out_type=jax.ShapeDtypeStruct((B, D), jnp.float32),
        scratch_types=[
            pltpu.VMEM((n_ch, _CH), jnp.int32),
            pltpu.VMEM((_CH, D), jnp.float32),
            pltpu.SemaphoreType.DMA,
        ],
        compiler_params=pltpu.CompilerParams(use_tc_tiling_on_sc=False),
    )
    def k(table_hbm, idx_hbm, out_hbm, idx_v, rows_v, gsem):
        wid = lax.axis_index("s") * _NC + lax.axis_index("c")
        base = wid * b_per_w
        pltpu.sync_copy(idx_hbm.at[wid], idx_v)

        def body(j, carry):
            pltpu.async_copy(table_hbm.at[idx_v.at[j]], rows_v, gsem).wait()
            pltpu.sync_copy(rows_v, out_hbm.at[pl.ds(base + j * _CH, _CH)])
            return carry

        lax.fori_loop(0, n_ch, body, 0, unroll=False)

    return k(table, idx3)


def kernel(x, W_E):
    B0, B1 = x.shape
    V, D = W_E.shape
    B = B0 * B1
    n_ch = B // (_NW * _CH)
    idx3 = x.reshape(_NW, n_ch, _CH).astype(jnp.int32)
    out = _embed_gather(W_E, idx3, B, D, n_ch)
    return out.reshape(B0, B1, D)


# speedup vs baseline: 1.4958x; 1.1447x over previous
import functools

import jax
import jax.numpy as jnp
from jax import lax
from jax.experimental import pallas as pl
from jax.experimental.pallas import tpu as pltpu
from jax.experimental.pallas import tpu_sc as plsc


_info = plsc.get_sparse_core_info()
_NC, _NS = _info.num_cores, _info.num_subcores
_NW = _NC * _NS

_CH = 128
_K = 8


def _embed_gather(table, idx3, B, D, n_ch):
    mesh = plsc.VectorSubcoreMesh(core_axis_name="c", subcore_axis_name="s")
    b_per_w = n_ch * _CH
    n_grp = n_ch // _K
    grp_rows = _K * _CH

    @functools.partial(
        pl.kernel,
        mesh=mesh,
        out_type=jax.ShapeDtypeStruct((B, D), jnp.float32),
        scratch_types=[
            pltpu.VMEM((n_ch, _CH), jnp.int32),
            pltpu.VMEM((2, grp_rows, D), jnp.float32),
            pltpu.SemaphoreType.DMA,
            pltpu.SemaphoreType.DMA,
        ],
        compiler_params=pltpu.CompilerParams(use_tc_tiling_on_sc=False),
    )
    def k(table_hbm, idx_hbm, out_hbm, idx_v, rows_v, gsem, osem):
        wid = lax.axis_index("s") * _NC + lax.axis_index("c")
        base = wid * b_per_w
        pltpu.sync_copy(idx_hbm.at[wid], idx_v)

        def fire_group(g, buf):
            for b in range(_K):
                pltpu.async_copy(
                    table_hbm.at[idx_v.at[g * _K + b]],
                    rows_v.at[buf, pl.ds(b * _CH, _CH)],
                    gsem,
                )

        def drain_group(g, buf):
            for b in range(_K):
                pltpu.make_async_copy(
                    table_hbm.at[idx_v.at[g * _K + b]],
                    rows_v.at[buf, pl.ds(b * _CH, _CH)],
                    gsem,
                ).wait()

        def out_start(g, buf):
            pltpu.async_copy(
                rows_v.at[buf], out_hbm.at[pl.ds(base + g * grp_rows, grp_rows)], osem
            )

        def out_wait(g, buf):
            pltpu.make_async_copy(
                rows_v.at[buf], out_hbm.at[pl.ds(base + g * grp_rows, grp_rows)], osem
            ).wait()

        fire_group(0, 0)

        def body(g, carry):
            buf = lax.rem(g, 2)
            drain_group(g, buf)

            @pl.when(g > 0)
            def _():
                out_wait(g - 1, 1 - buf)

            out_start(g, buf)

            @pl.when(g + 1 < n_grp)
            def _():
                fire_group(g + 1, 1 - buf)

            return carry

        lax.fori_loop(0, n_grp, body, 0, unroll=False)
        out_wait(n_grp - 1, lax.rem(n_grp - 1, 2))

    return k(table, idx3)


def kernel(x, W_E):
    B0, B1 = x.shape
    V, D = W_E.shape
    B = B0 * B1
    n_ch = B // (_NW * _CH)
    idx3 = x.reshape(_NW, n_ch, _CH).astype(jnp.int32)
    out = _embed_gather(W_E, idx3, B, D, n_ch)
    return out.reshape(B0, B1, D)
